# baseline (speedup 1.0000x reference)
import jax
import jax.numpy as jnp
from jax import lax
from jax.experimental import pallas as pl
from jax.experimental.pallas import tpu as pltpu

_C = 19
_SMOOTHING = 0.1
_CONFIDENCE = 1.0 - _SMOOTHING
_SV = _SMOOTHING / (_C - 1)
_IGNORE = 255

_B = 8
_P = 512 * 512
_CH = 2048
_NJ = _P // _CH
_SL = 8
_LN = _CH // _SL


def _tc_body(x_ref, t_ref, out_ref):
    b = pl.program_id(0)
    j = pl.program_id(1)

    @pl.when(jnp.logical_and(b == 0, j == 0))
    def _():
        out_ref[...] = jnp.zeros((_SL, _LN), jnp.float32)

    def tree(vals, op):
        while len(vals) > 1:
            nxt = [op(vals[i], vals[i + 1]) for i in range(0, len(vals) - 1, 2)]
            if len(vals) % 2:
                nxt.append(vals[-1])
            vals = nxt
        return vals[0]

    xs = [x_ref[0, c, 0] for c in range(_C)]
    t = t_ref[0, 0]
    m = tree(list(xs), jnp.maximum)
    sp = tree(list(xs), jnp.add)
    s = tree([jnp.exp(x - m) for x in xs], jnp.add)
    lse = m + jnp.log(s)
    mask = t != _IGNORE
    tcl = jnp.where(mask, t, 0)
    pt = tree([jnp.where(tcl == c, xs[c], 0.0) for c in range(_C)], jnp.add)
    val = jnp.where(mask, lse - _SV * sp - (_CONFIDENCE - _SV) * pt, 0.0)
    out_ref[...] += val


@jax.jit
def kernel(pred, target):
    pred5 = pred.reshape(_B, _C, _NJ, _SL, _LN)
    tgt4 = target.reshape(_B, _NJ, _SL, _LN)
    partial = pl.pallas_call(
        _tc_body,
        grid=(_B, _NJ),
        in_specs=[
            pl.BlockSpec((1, _C, 1, _SL, _LN), lambda b, j: (b, 0, j, 0, 0)),
            pl.BlockSpec((1, 1, _SL, _LN), lambda b, j: (b, j, 0, 0)),
        ],
        out_specs=pl.BlockSpec((_SL, _LN), lambda b, j: (0, 0)),
        out_shape=jax.ShapeDtypeStruct((_SL, _LN), jnp.float32),
    )(pred5, tgt4)
    return jnp.sum(partial) * (1.0 / (_B * _P))


# speedup vs baseline: 2.2482x; 2.2482x over previous
import jax
import jax.numpy as jnp
from jax import lax
from jax.experimental import pallas as pl
from jax.experimental.pallas import tpu as pltpu

_C = 19
_SMOOTHING = 0.1
_CONFIDENCE = 1.0 - _SMOOTHING
_SV = _SMOOTHING / (_C - 1)
_IGNORE = 255

_B = 8
_P = 512 * 512
_CH = 16384
_NJ = _P // _CH
_SL = 8
_LN = _CH // _SL


def _tc_body(x_ref, t_ref, out_ref):
    b = pl.program_id(0)
    j = pl.program_id(1)

    @pl.when(jnp.logical_and(b == 0, j == 0))
    def _():
        out_ref[...] = jnp.zeros((_SL, _LN), jnp.float32)

    t = t_ref[0, 0]
    m = x_ref[0, 0, 0]
    sp = x_ref[0, 0, 0]
    for c in range(1, _C):
        x = x_ref[0, c, 0]
        m = jnp.maximum(m, x)
        sp = sp + x
    s = jnp.exp(x_ref[0, 0, 0] - m)
    for c in range(1, _C):
        s = s + jnp.exp(x_ref[0, c, 0] - m)
    lse = m + jnp.log(s)
    mask = t != _IGNORE
    tcl = jnp.where(mask, t, 0)
    pt = jnp.where(tcl == 0, x_ref[0, 0, 0], 0.0)
    for c in range(1, _C):
        pt = pt + jnp.where(tcl == c, x_ref[0, c, 0], 0.0)
    val = jnp.where(mask, lse - _SV * sp - (_CONFIDENCE - _SV) * pt, 0.0)
    out_ref[...] += val


@jax.jit
def kernel(pred, target):
    pred5 = pred.reshape(_B, _C, _NJ, _SL, _LN)
    tgt4 = target.reshape(_B, _NJ, _SL, _LN)
    partial = pl.pallas_call(
        _tc_body,
        grid=(_B, _NJ),
        in_specs=[
            pl.BlockSpec((1, _C, 1, _SL, _LN), lambda b, j: (b, 0, j, 0, 0)),
            pl.BlockSpec((1, 1, _SL, _LN), lambda b, j: (b, j, 0, 0)),
        ],
        out_specs=pl.BlockSpec((_SL, _LN), lambda b, j: (0, 0)),
        out_shape=jax.ShapeDtypeStruct((_SL, _LN), jnp.float32),
    )(pred5, tgt4)
    return jnp.sum(partial) * (1.0 / (_B * _P))


# speedup vs baseline: 9.6668x; 4.2998x over previous
import jax
import jax.numpy as jnp
from jax import lax
from jax.experimental import pallas as pl
from jax.experimental.pallas import tpu as pltpu

_C = 19
_SMOOTHING = 0.1
_CONFIDENCE = 1.0 - _SMOOTHING
_SV = _SMOOTHING / (_C - 1)
_IGNORE = 255

_B = 8
_H = 512
_W = 512
_P = _H * _W
_HB = 128
_NJ = _H // _HB


def _tc_body(x_ref, t_ref, out_ref):
    b = pl.program_id(0)
    j = pl.program_id(1)

    @pl.when(jnp.logical_and(b == 0, j == 0))
    def _():
        out_ref[...] = jnp.zeros((_HB, _W), jnp.float32)

    t = t_ref[0]
    m = x_ref[0, 0]
    sp = x_ref[0, 0]
    for c in range(1, _C):
        x = x_ref[0, c]
        m = jnp.maximum(m, x)
        sp = sp + x
    s = jnp.exp(x_ref[0, 0] - m)
    for c in range(1, _C):
        s = s + jnp.exp(x_ref[0, c] - m)
    lse = m + jnp.log(s)
    mask = t != _IGNORE
    tcl = jnp.where(mask, t, 0)
    pt = jnp.where(tcl == 0, x_ref[0, 0], 0.0)
    for c in range(1, _C):
        pt = pt + jnp.where(tcl == c, x_ref[0, c], 0.0)
    val = jnp.where(mask, lse - _SV * sp - (_CONFIDENCE - _SV) * pt, 0.0)
    out_ref[...] += val


@jax.jit
def kernel(pred, target):
    partial = pl.pallas_call(
        _tc_body,
        grid=(_B, _NJ),
        in_specs=[
            pl.BlockSpec((1, _C, _HB, _W), lambda b, j: (b, 0, j, 0)),
            pl.BlockSpec((1, _HB, _W), lambda b, j: (b, j, 0)),
        ],
        out_specs=pl.BlockSpec((_HB, _W), lambda b, j: (0, 0)),
        out_shape=jax.ShapeDtypeStruct((_HB, _W), jnp.float32),
    )(pred, target)
    return jnp.sum(partial) * (1.0 / (_B * _P))
